# baseline (speedup 1.0000x reference)
import math

import numpy as np
import jax
import jax.numpy as jnp
from jax import lax
from jax.experimental import pallas as pl
from jax.experimental.pallas import tpu as pltpu
from jax.experimental.pallas import tpu_sc as plsc

N = 10000
E = 640000
H = 128
F = 64
G = 50
L = 3
B = 100
CUTOFF = 10.0
LOG2 = math.log(2.0)

NC, NS = 2, 16
NW = NC * NS
EW = E // NW
K = 80
NCH = EW // K
RPS = N // NS

TILE_N = 1000
GRID_N = N // TILE_N
GP = 64
SPACING = CUTOFF / (G - 1)
COEFF = -0.5 / SPACING ** 2

TBL = 32768
TILE_T = 2048
GRID_T = TBL // TILE_T
WMAX = float(np.sqrt(27.0) * (1.0 + 1e-6))
DW = WMAX / (TBL - 1)
TILE_Q = 8000
GRID_Q = E // TILE_Q

_HIGH = jax.lax.Precision.HIGHEST

_mesh = plsc.VectorSubcoreMesh(core_axis_name="c", subcore_axis_name="s",
                               num_cores=NC, num_subcores=NS)


def _ssp(x):
    return jax.nn.softplus(x) - LOG2


def _geom_body(px_hbm, py_hbm, pz_hbm, src_hbm, dst_hbm, out_hbm,
               px_v, py_v, pz_v, src_v, dst_v, o_v):
    cid = lax.axis_index("c")
    sid = lax.axis_index("s")
    wid = sid * NC + cid
    base = wid * EW
    pltpu.sync_copy(px_hbm, px_v)
    pltpu.sync_copy(py_hbm, py_v)
    pltpu.sync_copy(pz_hbm, pz_v)
    pltpu.sync_copy(src_hbm.at[pl.ds(base, EW)], src_v)
    pltpu.sync_copy(dst_hbm.at[pl.ds(base, EW)], dst_v)

    @pl.loop(0, EW // 16)
    def _(i):
        s = src_v[pl.ds(i * 16, 16)]
        d = dst_v[pl.ds(i * 16, 16)]
        dx = plsc.load_gather(px_v, [s]) - plsc.load_gather(px_v, [d])
        dy = plsc.load_gather(py_v, [s]) - plsc.load_gather(py_v, [d])
        dz = plsc.load_gather(pz_v, [s]) - plsc.load_gather(pz_v, [d])
        o_v[pl.ds(i * 16, 16)] = dx * dx + dy * dy + dz * dz

    pltpu.sync_copy(o_v, out_hbm.at[pl.ds(base, EW)])


_geom = pl.kernel(
    _geom_body,
    out_type=jax.ShapeDtypeStruct((E,), jnp.float32),
    mesh=_mesh,
    scratch_types=[
        pltpu.VMEM((N,), jnp.float32),
        pltpu.VMEM((N,), jnp.float32),
        pltpu.VMEM((N,), jnp.float32),
        pltpu.VMEM((EW,), jnp.int32),
        pltpu.VMEM((EW,), jnp.int32),
        pltpu.VMEM((EW,), jnp.float32),
    ],
    compiler_params=pltpu.CompilerParams(needs_layout_passes=False),
)


def _gms_body(xf_hbm, tbl_hbm, src_hbm, dst_hbm, qidx_hbm, zero_hbm, out_hbm,
              src2_v, dst2_v, q2_v, gath0, gath1, wf0, wf1, agg_sh,
              gsem0, gsem1, wsem0, wsem1):
    cid = lax.axis_index("c")
    sid = lax.axis_index("s")
    wid = sid * NC + cid
    pltpu.sync_copy(zero_hbm.at[pl.ds(sid * RPS, RPS)],
                    agg_sh.at[pl.ds(sid * RPS, RPS)])
    pltpu.sync_copy(src_hbm.at[wid], src2_v)
    pltpu.sync_copy(dst_hbm.at[wid], dst2_v)
    pltpu.sync_copy(qidx_hbm.at[wid], q2_v)
    plsc.subcore_barrier()

    def start(cc, gath, wfb, gs, ws):
        pltpu.async_copy(xf_hbm.at[src2_v.at[cc]], gath, gs)
        pltpu.async_copy(tbl_hbm.at[q2_v.at[cc]], wfb, ws)

    def finish(cc, gath, wfb, gs, ws):
        pltpu.make_async_copy(xf_hbm.at[src2_v.at[cc]], gath, gs).wait()
        pltpu.make_async_copy(tbl_hbm.at[q2_v.at[cc]], wfb, ws).wait()

        @pl.loop(0, K, unroll=4)
        def _(r):
            for c in range(4):
                gath[r, pl.ds(c * 16, 16)] = (
                    gath[r, pl.ds(c * 16, 16)] * wfb[r, pl.ds(c * 16, 16)])

        pltpu.sync_copy(gath, agg_sh.at[dst2_v.at[cc]], add=True)

    start(0, gath0, wf0, gsem0, wsem0)

    @pl.loop(0, NCH // 2)
    def _(j):
        cc0 = 2 * j
        start(cc0 + 1, gath1, wf1, gsem1, wsem1)
        finish(cc0, gath0, wf0, gsem0, wsem0)

        @pl.when(j < NCH // 2 - 1)
        def _():
            start(cc0 + 2, gath0, wf0, gsem0, wsem0)

        finish(cc0 + 1, gath1, wf1, gsem1, wsem1)

    plsc.subcore_barrier()
    pltpu.sync_copy(agg_sh.at[pl.ds(sid * RPS, RPS)],
                    out_hbm.at[pl.ds(cid * N + sid * RPS, RPS)])


_gms = pl.kernel(
    _gms_body,
    out_type=jax.ShapeDtypeStruct((2 * N, F), jnp.float32),
    mesh=_mesh,
    scratch_types=[
        pltpu.VMEM((NCH, K), jnp.int32),
        pltpu.VMEM((NCH, K), jnp.int32),
        pltpu.VMEM((NCH, K), jnp.int32),
        pltpu.VMEM((K, F), jnp.float32),
        pltpu.VMEM((K, F), jnp.float32),
        pltpu.VMEM((K, F), jnp.float32),
        pltpu.VMEM((K, F), jnp.float32),
        pltpu.VMEM_SHARED((N, F), jnp.float32),
        pltpu.SemaphoreType.DMA,
        pltpu.SemaphoreType.DMA,
        pltpu.SemaphoreType.DMA,
        pltpu.SemaphoreType.DMA,
    ],
    compiler_params=pltpu.CompilerParams(needs_layout_passes=False,
                                         use_tc_tiling_on_sc=False),
)


def _embed_body(z_ref, emb_ref, w1_ref, h_ref, xf_ref):
    zb = z_ref[0, 0, :]
    ids = lax.broadcasted_iota(jnp.int32, (TILE_N, 128), 1)
    oh = (zb[:, None] == ids).astype(jnp.float32)
    h = jnp.dot(oh, emb_ref[...], precision=_HIGH,
                preferred_element_type=jnp.float32)
    h_ref[...] = h
    xf_ref[...] = jnp.dot(h, w1_ref[...], precision=_HIGH,
                          preferred_element_type=jnp.float32)


_embed = pl.pallas_call(
    _embed_body,
    grid=(GRID_N,),
    in_specs=[
        pl.BlockSpec((1, 1, TILE_N), lambda i: (i, 0, 0)),
        pl.BlockSpec((128, H), lambda i: (0, 0)),
        pl.BlockSpec((H, F), lambda i: (0, 0)),
    ],
    out_specs=[
        pl.BlockSpec((TILE_N, H), lambda i: (i, 0)),
        pl.BlockSpec((TILE_N, F), lambda i: (i, 0)),
    ],
    out_shape=[
        jax.ShapeDtypeStruct((N, H), jnp.float32),
        jax.ShapeDtypeStruct((N, F), jnp.float32),
    ],
)


def _ftab_body(w1_ref, b1_ref, w2_ref, b2_ref, out_ref):
    t = pl.program_id(1)
    row = lax.broadcasted_iota(jnp.int32, (TILE_T, 1), 0) + t * TILE_T
    w = row.astype(jnp.float32) * DW
    offs = (lax.broadcasted_iota(jnp.int32, (1, GP), 1).astype(jnp.float32)
            * SPACING)
    attr = jnp.exp(COEFF * (w - offs) ** 2)
    tv = _ssp(jnp.dot(attr, w1_ref[0],
                      preferred_element_type=jnp.float32) + b1_ref[0])
    wf = jnp.dot(tv, w2_ref[0],
                 preferred_element_type=jnp.float32) + b2_ref[0]
    c = 0.5 * (jnp.cos(w * (math.pi / CUTOFF)) + 1.0)
    out_ref[0] = wf * c


_ftab = pl.pallas_call(
    _ftab_body,
    grid=(L, GRID_T),
    in_specs=[
        pl.BlockSpec((1, GP, F), lambda l, t: (l, 0, 0)),
        pl.BlockSpec((1, 1, F), lambda l, t: (l, 0, 0)),
        pl.BlockSpec((1, F, F), lambda l, t: (l, 0, 0)),
        pl.BlockSpec((1, 1, F), lambda l, t: (l, 0, 0)),
    ],
    out_specs=pl.BlockSpec((1, TILE_T, F), lambda l, t: (l, t, 0)),
    out_shape=jax.ShapeDtypeStruct((L, TBL, F), jnp.float32),
)


def _quant_body(wsq_ref, out_ref):
    w = jnp.sqrt(wsq_ref[0, 0, :] + 1e-12)
    q = jnp.round(w * (1.0 / DW)).astype(jnp.int32)
    out_ref[0, 0, :] = jnp.clip(q, 0, TBL - 1)


_quant = pl.pallas_call(
    _quant_body,
    grid=(GRID_Q,),
    in_specs=[pl.BlockSpec((1, 1, TILE_Q), lambda i: (i, 0, 0))],
    out_specs=pl.BlockSpec((1, 1, TILE_Q), lambda i: (i, 0, 0)),
    out_shape=jax.ShapeDtypeStruct((GRID_Q, 1, TILE_Q), jnp.int32),
)


def _update_body(agg_ref, h_ref, w2_ref, b2_ref, lw_ref, lb_ref, w1n_ref,
                 hout_ref, xfout_ref):
    agg = agg_ref[0] + agg_ref[1]
    x2 = jnp.dot(agg, w2_ref[...], precision=_HIGH,
                 preferred_element_type=jnp.float32) + b2_ref[...]
    hn = h_ref[...] + jnp.dot(_ssp(x2), lw_ref[...], precision=_HIGH,
                              preferred_element_type=jnp.float32) + lb_ref[...]
    hout_ref[...] = hn
    xfout_ref[...] = jnp.dot(hn, w1n_ref[...], precision=_HIGH,
                             preferred_element_type=jnp.float32)


_update = pl.pallas_call(
    _update_body,
    grid=(GRID_N,),
    in_specs=[
        pl.BlockSpec((2, TILE_N, F), lambda i: (0, i, 0)),
        pl.BlockSpec((TILE_N, H), lambda i: (i, 0)),
        pl.BlockSpec((F, H), lambda i: (0, 0)),
        pl.BlockSpec((1, H), lambda i: (0, 0)),
        pl.BlockSpec((H, H), lambda i: (0, 0)),
        pl.BlockSpec((1, H), lambda i: (0, 0)),
        pl.BlockSpec((H, F), lambda i: (0, 0)),
    ],
    out_specs=[
        pl.BlockSpec((TILE_N, H), lambda i: (i, 0)),
        pl.BlockSpec((TILE_N, F), lambda i: (i, 0)),
    ],
    out_shape=[
        jax.ShapeDtypeStruct((N, H), jnp.float32),
        jax.ShapeDtypeStruct((N, F), jnp.float32),
    ],
)


def _readout_body(h_ref, batch_ref, o1w_ref, o1b_ref, o2w_ref, o2b_ref,
                  out_ref, acc_s, cnt_s):
    i = pl.program_id(0)
    y = _ssp(jnp.dot(h_ref[...], o1w_ref[...], precision=_HIGH,
                     preferred_element_type=jnp.float32) + o1b_ref[...])
    y = jnp.dot(y, o2w_ref[...], precision=_HIGH,
                preferred_element_type=jnp.float32) + o2b_ref[...]
    bb = batch_ref[0, 0, :]
    rows = lax.broadcasted_iota(jnp.int32, (128, TILE_N), 0)
    mask = (rows == bb[None, :]).astype(jnp.float32)
    ms = jnp.dot(mask, y, precision=_HIGH, preferred_element_type=jnp.float32)
    mc = jnp.dot(mask, jnp.ones((TILE_N, 128), jnp.float32), precision=_HIGH,
                 preferred_element_type=jnp.float32)

    @pl.when(i == 0)
    def _():
        acc_s[...] = ms
        cnt_s[...] = mc

    @pl.when(i > 0)
    def _():
        acc_s[...] += ms
        cnt_s[...] += mc

    @pl.when(i == GRID_N - 1)
    def _():
        out_ref[...] = acc_s[...] / jnp.maximum(cnt_s[...], 1.0)


_readout = pl.pallas_call(
    _readout_body,
    grid=(GRID_N,),
    in_specs=[
        pl.BlockSpec((TILE_N, H), lambda i: (i, 0)),
        pl.BlockSpec((1, 1, TILE_N), lambda i: (i, 0, 0)),
        pl.BlockSpec((H, H), lambda i: (0, 0)),
        pl.BlockSpec((1, H), lambda i: (0, 0)),
        pl.BlockSpec((H, H), lambda i: (0, 0)),
        pl.BlockSpec((1, H), lambda i: (0, 0)),
    ],
    out_specs=pl.BlockSpec((128, H), lambda i: (0, 0)),
    out_shape=jax.ShapeDtypeStruct((128, H), jnp.float32),
    scratch_shapes=[
        pltpu.VMEM((128, H), jnp.float32),
        pltpu.VMEM((128, H), jnp.float32),
    ],
)


def kernel(z, pos, edge_index, batch, emb, mlp_w1, mlp_b1, mlp_w2, mlp_b2,
           cf_w1, cf_w2, cf_b2, lin_w, lin_b, out1_w, out1_b, out2_w, out2_b):
    src = edge_index[0]
    dst = edge_index[1]
    wsq = _geom(pos[:, 0], pos[:, 1], pos[:, 2], src, dst)
    embp = jnp.zeros((128, H), jnp.float32).at[:120].set(emb)
    h, xf = _embed(z.reshape(GRID_N, 1, TILE_N), embp, cf_w1[0])
    src2 = src.reshape(NW, NCH, K)
    dst2 = dst.reshape(NW, NCH, K)
    zeros = jnp.zeros((N, F), jnp.float32)
    qidx = _quant(wsq.reshape(GRID_Q, 1, TILE_Q)).reshape(NW, NCH, K)
    w1p = jnp.zeros((L, GP, F), jnp.float32).at[:, :G].set(mlp_w1)
    tbl = _ftab(w1p, mlp_b1.reshape(L, 1, F), mlp_w2,
                mlp_b2.reshape(L, 1, F))
    for l in range(L):
        agg2 = _gms(xf, tbl[l], src2, dst2, qidx, zeros)
        h, xf = _update(agg2.reshape(2, N, F), h, cf_w2[l],
                        cf_b2[l].reshape(1, H), lin_w[l], lin_b[l].reshape(1, H),
                        cf_w1[(l + 1) % L])
    out = _readout(h, batch.reshape(GRID_N, 1, TILE_N), out1_w,
                   out1_b.reshape(1, H), out2_w, out2_b.reshape(1, H))
    return out[:B]


# speedup vs baseline: 1.5544x; 1.5544x over previous
import math

import numpy as np
import jax
import jax.numpy as jnp
from jax import lax
from jax.experimental import pallas as pl
from jax.experimental.pallas import tpu as pltpu
from jax.experimental.pallas import tpu_sc as plsc

N = 10000
E = 640000
H = 128
F = 64
G = 50
L = 3
B = 100
CUTOFF = 10.0
LOG2 = math.log(2.0)

NC, NS = 2, 16
NW = NC * NS
EW = E // NW
K = 80
NCH = EW // K
RPS = N // NS

TILE_N = 1000
GRID_N = N // TILE_N
GP = 64
SPACING = CUTOFF / (G - 1)
COEFF = -0.5 / SPACING ** 2

TBL = 32768
TILE_T = 2048
GRID_T = TBL // TILE_T
WMAX = float(np.sqrt(27.0) * (1.0 + 1e-6))
DW = WMAX / (TBL - 1)
TILE_Q = 8000
GRID_Q = E // TILE_Q

_HIGH = jax.lax.Precision.HIGHEST

_mesh = plsc.VectorSubcoreMesh(core_axis_name="c", subcore_axis_name="s",
                               num_cores=NC, num_subcores=NS)


def _ssp(x):
    return jax.nn.softplus(x) - LOG2


def _geom_body(px_hbm, py_hbm, pz_hbm, src_hbm, dst_hbm, out_hbm,
               px_v, py_v, pz_v, src_v, dst_v, o_v):
    cid = lax.axis_index("c")
    sid = lax.axis_index("s")
    wid = sid * NC + cid
    base = wid * EW
    pltpu.sync_copy(px_hbm, px_v)
    pltpu.sync_copy(py_hbm, py_v)
    pltpu.sync_copy(pz_hbm, pz_v)
    pltpu.sync_copy(src_hbm.at[pl.ds(base, EW)], src_v)
    pltpu.sync_copy(dst_hbm.at[pl.ds(base, EW)], dst_v)

    @pl.loop(0, EW // 16)
    def _(i):
        s = src_v[pl.ds(i * 16, 16)]
        d = dst_v[pl.ds(i * 16, 16)]
        dx = plsc.load_gather(px_v, [s]) - plsc.load_gather(px_v, [d])
        dy = plsc.load_gather(py_v, [s]) - plsc.load_gather(py_v, [d])
        dz = plsc.load_gather(pz_v, [s]) - plsc.load_gather(pz_v, [d])
        o_v[pl.ds(i * 16, 16)] = dx * dx + dy * dy + dz * dz

    pltpu.sync_copy(o_v, out_hbm.at[pl.ds(base, EW)])


_geom = pl.kernel(
    _geom_body,
    out_type=jax.ShapeDtypeStruct((E,), jnp.float32),
    mesh=_mesh,
    scratch_types=[
        pltpu.VMEM((N,), jnp.float32),
        pltpu.VMEM((N,), jnp.float32),
        pltpu.VMEM((N,), jnp.float32),
        pltpu.VMEM((EW,), jnp.int32),
        pltpu.VMEM((EW,), jnp.int32),
        pltpu.VMEM((EW,), jnp.float32),
    ],
    compiler_params=pltpu.CompilerParams(needs_layout_passes=False),
)


def _gms_body(xf_hbm, tbl_hbm, src_hbm, dst_hbm, qidx_hbm, zero_hbm, out_hbm,
              src2_v, dst2_v, q2_v, gath0, gath1, wf0, wf1, agg_sh,
              gsem0, gsem1, wsem0, wsem1):
    cid = lax.axis_index("c")
    sid = lax.axis_index("s")
    wid = sid * NC + cid
    pltpu.sync_copy(zero_hbm.at[pl.ds(sid * RPS, RPS)],
                    agg_sh.at[pl.ds(sid * RPS, RPS)])
    pltpu.sync_copy(src_hbm.at[wid], src2_v)
    pltpu.sync_copy(dst_hbm.at[wid], dst2_v)
    pltpu.sync_copy(qidx_hbm.at[wid], q2_v)
    plsc.subcore_barrier()

    def start(cc, gath, wfb, gs, ws):
        pltpu.async_copy(xf_hbm.at[src2_v.at[cc]], gath, gs)
        pltpu.async_copy(tbl_hbm.at[q2_v.at[cc]], wfb, ws)

    def finish(cc, gath, wfb, gs, ws):
        pltpu.make_async_copy(xf_hbm.at[src2_v.at[cc]], gath, gs).wait()
        pltpu.make_async_copy(tbl_hbm.at[q2_v.at[cc]], wfb, ws).wait()

        @pl.loop(0, K)
        def _(r):
            for c in range(4):
                gath[r, pl.ds(c * 16, 16)] = (
                    gath[r, pl.ds(c * 16, 16)] * wfb[r, pl.ds(c * 16, 16)])

        pltpu.sync_copy(gath, agg_sh.at[dst2_v.at[cc]], add=True)

    start(0, gath0, wf0, gsem0, wsem0)

    @pl.loop(0, NCH // 2)
    def _(j):
        cc0 = 2 * j
        start(cc0 + 1, gath1, wf1, gsem1, wsem1)
        finish(cc0, gath0, wf0, gsem0, wsem0)

        @pl.when(j < NCH // 2 - 1)
        def _():
            start(cc0 + 2, gath0, wf0, gsem0, wsem0)

        finish(cc0 + 1, gath1, wf1, gsem1, wsem1)

    plsc.subcore_barrier()
    pltpu.sync_copy(agg_sh.at[pl.ds(sid * RPS, RPS)],
                    out_hbm.at[pl.ds(cid * N + sid * RPS, RPS)])


_gms = pl.kernel(
    _gms_body,
    out_type=jax.ShapeDtypeStruct((2 * N, F), jnp.float32),
    mesh=_mesh,
    scratch_types=[
        pltpu.VMEM((NCH, K), jnp.int32),
        pltpu.VMEM((NCH, K), jnp.int32),
        pltpu.VMEM((NCH, K), jnp.int32),
        pltpu.VMEM((K, F), jnp.float32),
        pltpu.VMEM((K, F), jnp.float32),
        pltpu.VMEM((K, F), jnp.float32),
        pltpu.VMEM((K, F), jnp.float32),
        pltpu.VMEM_SHARED((N, F), jnp.float32),
        pltpu.SemaphoreType.DMA,
        pltpu.SemaphoreType.DMA,
        pltpu.SemaphoreType.DMA,
        pltpu.SemaphoreType.DMA,
    ],
    compiler_params=pltpu.CompilerParams(needs_layout_passes=False,
                                         use_tc_tiling_on_sc=False),
)


def _embed_body(z_ref, emb_ref, w1_ref, h_ref, xf_ref):
    zb = z_ref[0, 0, :]
    ids = lax.broadcasted_iota(jnp.int32, (TILE_N, 128), 1)
    oh = (zb[:, None] == ids).astype(jnp.float32)
    h = jnp.dot(oh, emb_ref[...], precision=_HIGH,
                preferred_element_type=jnp.float32)
    h_ref[...] = h
    xf_ref[...] = jnp.dot(h, w1_ref[...], precision=_HIGH,
                          preferred_element_type=jnp.float32)


_embed = pl.pallas_call(
    _embed_body,
    grid=(GRID_N,),
    in_specs=[
        pl.BlockSpec((1, 1, TILE_N), lambda i: (i, 0, 0)),
        pl.BlockSpec((128, H), lambda i: (0, 0)),
        pl.BlockSpec((H, F), lambda i: (0, 0)),
    ],
    out_specs=[
        pl.BlockSpec((TILE_N, H), lambda i: (i, 0)),
        pl.BlockSpec((TILE_N, F), lambda i: (i, 0)),
    ],
    out_shape=[
        jax.ShapeDtypeStruct((N, H), jnp.float32),
        jax.ShapeDtypeStruct((N, F), jnp.float32),
    ],
)


def _ftab_body(w1_ref, b1_ref, w2_ref, b2_ref, out_ref):
    t = pl.program_id(1)
    row = lax.broadcasted_iota(jnp.int32, (TILE_T, 1), 0) + t * TILE_T
    w = row.astype(jnp.float32) * DW
    offs = (lax.broadcasted_iota(jnp.int32, (1, GP), 1).astype(jnp.float32)
            * SPACING)
    attr = jnp.exp(COEFF * (w - offs) ** 2)
    tv = _ssp(jnp.dot(attr, w1_ref[0],
                      preferred_element_type=jnp.float32) + b1_ref[0])
    wf = jnp.dot(tv, w2_ref[0],
                 preferred_element_type=jnp.float32) + b2_ref[0]
    c = 0.5 * (jnp.cos(w * (math.pi / CUTOFF)) + 1.0)
    out_ref[0] = wf * c


_ftab = pl.pallas_call(
    _ftab_body,
    grid=(L, GRID_T),
    in_specs=[
        pl.BlockSpec((1, GP, F), lambda l, t: (l, 0, 0)),
        pl.BlockSpec((1, 1, F), lambda l, t: (l, 0, 0)),
        pl.BlockSpec((1, F, F), lambda l, t: (l, 0, 0)),
        pl.BlockSpec((1, 1, F), lambda l, t: (l, 0, 0)),
    ],
    out_specs=pl.BlockSpec((1, TILE_T, F), lambda l, t: (l, t, 0)),
    out_shape=jax.ShapeDtypeStruct((L, TBL, F), jnp.float32),
)


def _quant_body(wsq_ref, out_ref):
    w = jnp.sqrt(wsq_ref[0, 0, :] + 1e-12)
    q = jnp.round(w * (1.0 / DW)).astype(jnp.int32)
    out_ref[0, 0, :] = jnp.clip(q, 0, TBL - 1)


_quant = pl.pallas_call(
    _quant_body,
    grid=(GRID_Q,),
    in_specs=[pl.BlockSpec((1, 1, TILE_Q), lambda i: (i, 0, 0))],
    out_specs=pl.BlockSpec((1, 1, TILE_Q), lambda i: (i, 0, 0)),
    out_shape=jax.ShapeDtypeStruct((GRID_Q, 1, TILE_Q), jnp.int32),
)


def _update_body(agg_ref, h_ref, w2_ref, b2_ref, lw_ref, lb_ref, w1n_ref,
                 hout_ref, xfout_ref):
    agg = agg_ref[0] + agg_ref[1]
    x2 = jnp.dot(agg, w2_ref[...], precision=_HIGH,
                 preferred_element_type=jnp.float32) + b2_ref[...]
    hn = h_ref[...] + jnp.dot(_ssp(x2), lw_ref[...], precision=_HIGH,
                              preferred_element_type=jnp.float32) + lb_ref[...]
    hout_ref[...] = hn
    xfout_ref[...] = jnp.dot(hn, w1n_ref[...], precision=_HIGH,
                             preferred_element_type=jnp.float32)


_update = pl.pallas_call(
    _update_body,
    grid=(GRID_N,),
    in_specs=[
        pl.BlockSpec((2, TILE_N, F), lambda i: (0, i, 0)),
        pl.BlockSpec((TILE_N, H), lambda i: (i, 0)),
        pl.BlockSpec((F, H), lambda i: (0, 0)),
        pl.BlockSpec((1, H), lambda i: (0, 0)),
        pl.BlockSpec((H, H), lambda i: (0, 0)),
        pl.BlockSpec((1, H), lambda i: (0, 0)),
        pl.BlockSpec((H, F), lambda i: (0, 0)),
    ],
    out_specs=[
        pl.BlockSpec((TILE_N, H), lambda i: (i, 0)),
        pl.BlockSpec((TILE_N, F), lambda i: (i, 0)),
    ],
    out_shape=[
        jax.ShapeDtypeStruct((N, H), jnp.float32),
        jax.ShapeDtypeStruct((N, F), jnp.float32),
    ],
)


def _readout_body(h_ref, batch_ref, o1w_ref, o1b_ref, o2w_ref, o2b_ref,
                  out_ref, acc_s, cnt_s):
    i = pl.program_id(0)
    y = _ssp(jnp.dot(h_ref[...], o1w_ref[...], precision=_HIGH,
                     preferred_element_type=jnp.float32) + o1b_ref[...])
    y = jnp.dot(y, o2w_ref[...], precision=_HIGH,
                preferred_element_type=jnp.float32) + o2b_ref[...]
    bb = batch_ref[0, 0, :]
    rows = lax.broadcasted_iota(jnp.int32, (128, TILE_N), 0)
    mask = (rows == bb[None, :]).astype(jnp.float32)
    ms = jnp.dot(mask, y, precision=_HIGH, preferred_element_type=jnp.float32)
    mc = jnp.dot(mask, jnp.ones((TILE_N, 128), jnp.float32), precision=_HIGH,
                 preferred_element_type=jnp.float32)

    @pl.when(i == 0)
    def _():
        acc_s[...] = ms
        cnt_s[...] = mc

    @pl.when(i > 0)
    def _():
        acc_s[...] += ms
        cnt_s[...] += mc

    @pl.when(i == GRID_N - 1)
    def _():
        out_ref[...] = acc_s[...] / jnp.maximum(cnt_s[...], 1.0)


_readout = pl.pallas_call(
    _readout_body,
    grid=(GRID_N,),
    in_specs=[
        pl.BlockSpec((TILE_N, H), lambda i: (i, 0)),
        pl.BlockSpec((1, 1, TILE_N), lambda i: (i, 0, 0)),
        pl.BlockSpec((H, H), lambda i: (0, 0)),
        pl.BlockSpec((1, H), lambda i: (0, 0)),
        pl.BlockSpec((H, H), lambda i: (0, 0)),
        pl.BlockSpec((1, H), lambda i: (0, 0)),
    ],
    out_specs=pl.BlockSpec((128, H), lambda i: (0, 0)),
    out_shape=jax.ShapeDtypeStruct((128, H), jnp.float32),
    scratch_shapes=[
        pltpu.VMEM((128, H), jnp.float32),
        pltpu.VMEM((128, H), jnp.float32),
    ],
)


def kernel(z, pos, edge_index, batch, emb, mlp_w1, mlp_b1, mlp_w2, mlp_b2,
           cf_w1, cf_w2, cf_b2, lin_w, lin_b, out1_w, out1_b, out2_w, out2_b):
    src = edge_index[0]
    dst = edge_index[1]
    wsq = _geom(pos[:, 0], pos[:, 1], pos[:, 2], src, dst)
    embp = jnp.zeros((128, H), jnp.float32).at[:120].set(emb)
    h, xf = _embed(z.reshape(GRID_N, 1, TILE_N), embp, cf_w1[0])
    src2 = src.reshape(NW, NCH, K)
    dst2 = dst.reshape(NW, NCH, K)
    zeros = jnp.zeros((N, F), jnp.float32)
    qidx = _quant(wsq.reshape(GRID_Q, 1, TILE_Q)).reshape(NW, NCH, K)
    w1p = jnp.zeros((L, GP, F), jnp.float32).at[:, :G].set(mlp_w1)
    tbl = _ftab(w1p, mlp_b1.reshape(L, 1, F), mlp_w2,
                mlp_b2.reshape(L, 1, F))
    for l in range(L):
        agg2 = _gms(xf, tbl[l], src2, dst2, qidx, zeros)
        h, xf = _update(agg2.reshape(2, N, F), h, cf_w2[l],
                        cf_b2[l].reshape(1, H), lin_w[l], lin_b[l].reshape(1, H),
                        cf_w1[(l + 1) % L])
    out = _readout(h, batch.reshape(GRID_N, 1, TILE_N), out1_w,
                   out1_b.reshape(1, H), out2_w, out2_b.reshape(1, H))
    return out[:B]


# speedup vs baseline: 1.5564x; 1.0013x over previous
import math

import numpy as np
import jax
import jax.numpy as jnp
from jax import lax
from jax.experimental import pallas as pl
from jax.experimental.pallas import tpu as pltpu
from jax.experimental.pallas import tpu_sc as plsc

N = 10000
E = 640000
H = 128
F = 64
G = 50
L = 3
B = 100
CUTOFF = 10.0
LOG2 = math.log(2.0)

NC, NS = 2, 16
NW = NC * NS
EW = E // NW
K = 80
NCH = EW // K
RPS = N // NS

TILE_N = 1000
GRID_N = N // TILE_N
GP = 64
SPACING = CUTOFF / (G - 1)
COEFF = -0.5 / SPACING ** 2

TBL = 32768
TILE_T = 2048
GRID_T = TBL // TILE_T
WMAX = float(np.sqrt(27.0) * (1.0 + 1e-6))
DW = WMAX / (TBL - 1)
TILE_Q = 8000
GRID_Q = E // TILE_Q

_HIGH = jax.lax.Precision.HIGHEST

_mesh = plsc.VectorSubcoreMesh(core_axis_name="c", subcore_axis_name="s",
                               num_cores=NC, num_subcores=NS)


def _ssp(x):
    return jax.nn.softplus(x) - LOG2


def _geom_body(px_hbm, py_hbm, pz_hbm, src_hbm, dst_hbm, out_hbm,
               px_v, py_v, pz_v, src_v, dst_v, o_v):
    cid = lax.axis_index("c")
    sid = lax.axis_index("s")
    wid = sid * NC + cid
    base = wid * EW
    pltpu.sync_copy(px_hbm, px_v)
    pltpu.sync_copy(py_hbm, py_v)
    pltpu.sync_copy(pz_hbm, pz_v)
    pltpu.sync_copy(src_hbm.at[pl.ds(base, EW)], src_v)
    pltpu.sync_copy(dst_hbm.at[pl.ds(base, EW)], dst_v)

    @pl.loop(0, EW // 16)
    def _(i):
        s = src_v[pl.ds(i * 16, 16)]
        d = dst_v[pl.ds(i * 16, 16)]
        dx = plsc.load_gather(px_v, [s]) - plsc.load_gather(px_v, [d])
        dy = plsc.load_gather(py_v, [s]) - plsc.load_gather(py_v, [d])
        dz = plsc.load_gather(pz_v, [s]) - plsc.load_gather(pz_v, [d])
        o_v[pl.ds(i * 16, 16)] = dx * dx + dy * dy + dz * dz

    pltpu.sync_copy(o_v, out_hbm.at[pl.ds(base, EW)])


_geom = pl.kernel(
    _geom_body,
    out_type=jax.ShapeDtypeStruct((E,), jnp.float32),
    mesh=_mesh,
    scratch_types=[
        pltpu.VMEM((N,), jnp.float32),
        pltpu.VMEM((N,), jnp.float32),
        pltpu.VMEM((N,), jnp.float32),
        pltpu.VMEM((EW,), jnp.int32),
        pltpu.VMEM((EW,), jnp.int32),
        pltpu.VMEM((EW,), jnp.float32),
    ],
    compiler_params=pltpu.CompilerParams(needs_layout_passes=False),
)


def _gms_body(xf_hbm, tbl_hbm, src_hbm, dst_hbm, qidx_hbm, zero_hbm, out_hbm,
              src2_v, dst2_v, q2_v, gath0, gath1, wf0, wf1, agg_sh,
              gsem0, gsem1, wsem0, wsem1):
    cid = lax.axis_index("c")
    sid = lax.axis_index("s")
    wid = sid * NC + cid
    pltpu.sync_copy(zero_hbm.at[pl.ds(sid * RPS, RPS)],
                    agg_sh.at[pl.ds(sid * RPS, RPS)])
    pltpu.sync_copy(src_hbm.at[wid], src2_v)
    pltpu.sync_copy(dst_hbm.at[wid], dst2_v)
    pltpu.sync_copy(qidx_hbm.at[wid], q2_v)
    plsc.subcore_barrier()

    def start(cc, gath, wfb, gs, ws):
        pltpu.async_copy(xf_hbm.at[src2_v.at[cc]], gath, gs)
        pltpu.async_copy(tbl_hbm.at[q2_v.at[cc]], wfb, ws)

    def finish(cc, gath, wfb, gs, ws):
        pltpu.make_async_copy(xf_hbm.at[src2_v.at[cc]], gath, gs).wait()
        pltpu.make_async_copy(tbl_hbm.at[q2_v.at[cc]], wfb, ws).wait()

        @plsc.parallel_loop(0, K, 1)
        def _(r):
            for c in range(4):
                gath[r, pl.ds(c * 16, 16)] = (
                    gath[r, pl.ds(c * 16, 16)] * wfb[r, pl.ds(c * 16, 16)])

        pltpu.sync_copy(gath, agg_sh.at[dst2_v.at[cc]], add=True)

    start(0, gath0, wf0, gsem0, wsem0)

    @pl.loop(0, NCH // 2)
    def _(j):
        cc0 = 2 * j
        start(cc0 + 1, gath1, wf1, gsem1, wsem1)
        finish(cc0, gath0, wf0, gsem0, wsem0)

        @pl.when(j < NCH // 2 - 1)
        def _():
            start(cc0 + 2, gath0, wf0, gsem0, wsem0)

        finish(cc0 + 1, gath1, wf1, gsem1, wsem1)

    plsc.subcore_barrier()
    pltpu.sync_copy(agg_sh.at[pl.ds(sid * RPS, RPS)],
                    out_hbm.at[pl.ds(cid * N + sid * RPS, RPS)])


_gms = pl.kernel(
    _gms_body,
    out_type=jax.ShapeDtypeStruct((2 * N, F), jnp.float32),
    mesh=_mesh,
    scratch_types=[
        pltpu.VMEM((NCH, K), jnp.int32),
        pltpu.VMEM((NCH, K), jnp.int32),
        pltpu.VMEM((NCH, K), jnp.int32),
        pltpu.VMEM((K, F), jnp.float32),
        pltpu.VMEM((K, F), jnp.float32),
        pltpu.VMEM((K, F), jnp.float32),
        pltpu.VMEM((K, F), jnp.float32),
        pltpu.VMEM_SHARED((N, F), jnp.float32),
        pltpu.SemaphoreType.DMA,
        pltpu.SemaphoreType.DMA,
        pltpu.SemaphoreType.DMA,
        pltpu.SemaphoreType.DMA,
    ],
    compiler_params=pltpu.CompilerParams(needs_layout_passes=False,
                                         use_tc_tiling_on_sc=False),
)


def _embed_body(z_ref, emb_ref, w1_ref, h_ref, xf_ref):
    zb = z_ref[0, 0, :]
    ids = lax.broadcasted_iota(jnp.int32, (TILE_N, 128), 1)
    oh = (zb[:, None] == ids).astype(jnp.float32)
    h = jnp.dot(oh, emb_ref[...], precision=_HIGH,
                preferred_element_type=jnp.float32)
    h_ref[...] = h
    xf_ref[...] = jnp.dot(h, w1_ref[...], precision=_HIGH,
                          preferred_element_type=jnp.float32)


_embed = pl.pallas_call(
    _embed_body,
    grid=(GRID_N,),
    in_specs=[
        pl.BlockSpec((1, 1, TILE_N), lambda i: (i, 0, 0)),
        pl.BlockSpec((128, H), lambda i: (0, 0)),
        pl.BlockSpec((H, F), lambda i: (0, 0)),
    ],
    out_specs=[
        pl.BlockSpec((TILE_N, H), lambda i: (i, 0)),
        pl.BlockSpec((TILE_N, F), lambda i: (i, 0)),
    ],
    out_shape=[
        jax.ShapeDtypeStruct((N, H), jnp.float32),
        jax.ShapeDtypeStruct((N, F), jnp.float32),
    ],
)


def _ftab_body(w1_ref, b1_ref, w2_ref, b2_ref, out_ref):
    t = pl.program_id(1)
    row = lax.broadcasted_iota(jnp.int32, (TILE_T, 1), 0) + t * TILE_T
    w = row.astype(jnp.float32) * DW
    offs = (lax.broadcasted_iota(jnp.int32, (1, GP), 1).astype(jnp.float32)
            * SPACING)
    attr = jnp.exp(COEFF * (w - offs) ** 2)
    tv = _ssp(jnp.dot(attr, w1_ref[0],
                      preferred_element_type=jnp.float32) + b1_ref[0])
    wf = jnp.dot(tv, w2_ref[0],
                 preferred_element_type=jnp.float32) + b2_ref[0]
    c = 0.5 * (jnp.cos(w * (math.pi / CUTOFF)) + 1.0)
    out_ref[0] = wf * c


_ftab = pl.pallas_call(
    _ftab_body,
    grid=(L, GRID_T),
    in_specs=[
        pl.BlockSpec((1, GP, F), lambda l, t: (l, 0, 0)),
        pl.BlockSpec((1, 1, F), lambda l, t: (l, 0, 0)),
        pl.BlockSpec((1, F, F), lambda l, t: (l, 0, 0)),
        pl.BlockSpec((1, 1, F), lambda l, t: (l, 0, 0)),
    ],
    out_specs=pl.BlockSpec((1, TILE_T, F), lambda l, t: (l, t, 0)),
    out_shape=jax.ShapeDtypeStruct((L, TBL, F), jnp.float32),
)


def _quant_body(wsq_ref, out_ref):
    w = jnp.sqrt(wsq_ref[0, 0, :] + 1e-12)
    q = jnp.round(w * (1.0 / DW)).astype(jnp.int32)
    out_ref[0, 0, :] = jnp.clip(q, 0, TBL - 1)


_quant = pl.pallas_call(
    _quant_body,
    grid=(GRID_Q,),
    in_specs=[pl.BlockSpec((1, 1, TILE_Q), lambda i: (i, 0, 0))],
    out_specs=pl.BlockSpec((1, 1, TILE_Q), lambda i: (i, 0, 0)),
    out_shape=jax.ShapeDtypeStruct((GRID_Q, 1, TILE_Q), jnp.int32),
)


def _update_body(agg_ref, h_ref, w2_ref, b2_ref, lw_ref, lb_ref, w1n_ref,
                 hout_ref, xfout_ref):
    agg = agg_ref[0] + agg_ref[1]
    x2 = jnp.dot(agg, w2_ref[...], precision=_HIGH,
                 preferred_element_type=jnp.float32) + b2_ref[...]
    hn = h_ref[...] + jnp.dot(_ssp(x2), lw_ref[...], precision=_HIGH,
                              preferred_element_type=jnp.float32) + lb_ref[...]
    hout_ref[...] = hn
    xfout_ref[...] = jnp.dot(hn, w1n_ref[...], precision=_HIGH,
                             preferred_element_type=jnp.float32)


_update = pl.pallas_call(
    _update_body,
    grid=(GRID_N,),
    in_specs=[
        pl.BlockSpec((2, TILE_N, F), lambda i: (0, i, 0)),
        pl.BlockSpec((TILE_N, H), lambda i: (i, 0)),
        pl.BlockSpec((F, H), lambda i: (0, 0)),
        pl.BlockSpec((1, H), lambda i: (0, 0)),
        pl.BlockSpec((H, H), lambda i: (0, 0)),
        pl.BlockSpec((1, H), lambda i: (0, 0)),
        pl.BlockSpec((H, F), lambda i: (0, 0)),
    ],
    out_specs=[
        pl.BlockSpec((TILE_N, H), lambda i: (i, 0)),
        pl.BlockSpec((TILE_N, F), lambda i: (i, 0)),
    ],
    out_shape=[
        jax.ShapeDtypeStruct((N, H), jnp.float32),
        jax.ShapeDtypeStruct((N, F), jnp.float32),
    ],
)


def _readout_body(h_ref, batch_ref, o1w_ref, o1b_ref, o2w_ref, o2b_ref,
                  out_ref, acc_s, cnt_s):
    i = pl.program_id(0)
    y = _ssp(jnp.dot(h_ref[...], o1w_ref[...], precision=_HIGH,
                     preferred_element_type=jnp.float32) + o1b_ref[...])
    y = jnp.dot(y, o2w_ref[...], precision=_HIGH,
                preferred_element_type=jnp.float32) + o2b_ref[...]
    bb = batch_ref[0, 0, :]
    rows = lax.broadcasted_iota(jnp.int32, (128, TILE_N), 0)
    mask = (rows == bb[None, :]).astype(jnp.float32)
    ms = jnp.dot(mask, y, precision=_HIGH, preferred_element_type=jnp.float32)
    mc = jnp.dot(mask, jnp.ones((TILE_N, 128), jnp.float32), precision=_HIGH,
                 preferred_element_type=jnp.float32)

    @pl.when(i == 0)
    def _():
        acc_s[...] = ms
        cnt_s[...] = mc

    @pl.when(i > 0)
    def _():
        acc_s[...] += ms
        cnt_s[...] += mc

    @pl.when(i == GRID_N - 1)
    def _():
        out_ref[...] = acc_s[...] / jnp.maximum(cnt_s[...], 1.0)


_readout = pl.pallas_call(
    _readout_body,
    grid=(GRID_N,),
    in_specs=[
        pl.BlockSpec((TILE_N, H), lambda i: (i, 0)),
        pl.BlockSpec((1, 1, TILE_N), lambda i: (i, 0, 0)),
        pl.BlockSpec((H, H), lambda i: (0, 0)),
        pl.BlockSpec((1, H), lambda i: (0, 0)),
        pl.BlockSpec((H, H), lambda i: (0, 0)),
        pl.BlockSpec((1, H), lambda i: (0, 0)),
    ],
    out_specs=pl.BlockSpec((128, H), lambda i: (0, 0)),
    out_shape=jax.ShapeDtypeStruct((128, H), jnp.float32),
    scratch_shapes=[
        pltpu.VMEM((128, H), jnp.float32),
        pltpu.VMEM((128, H), jnp.float32),
    ],
)


def kernel(z, pos, edge_index, batch, emb, mlp_w1, mlp_b1, mlp_w2, mlp_b2,
           cf_w1, cf_w2, cf_b2, lin_w, lin_b, out1_w, out1_b, out2_w, out2_b):
    src = edge_index[0]
    dst = edge_index[1]
    wsq = _geom(pos[:, 0], pos[:, 1], pos[:, 2], src, dst)
    embp = jnp.zeros((128, H), jnp.float32).at[:120].set(emb)
    h, xf = _embed(z.reshape(GRID_N, 1, TILE_N), embp, cf_w1[0])
    src2 = src.reshape(NW, NCH, K)
    dst2 = dst.reshape(NW, NCH, K)
    zeros = jnp.zeros((N, F), jnp.float32)
    qidx = _quant(wsq.reshape(GRID_Q, 1, TILE_Q)).reshape(NW, NCH, K)
    w1p = jnp.zeros((L, GP, F), jnp.float32).at[:, :G].set(mlp_w1)
    tbl = _ftab(w1p, mlp_b1.reshape(L, 1, F), mlp_w2,
                mlp_b2.reshape(L, 1, F))
    for l in range(L):
        agg2 = _gms(xf, tbl[l], src2, dst2, qidx, zeros)
        h, xf = _update(agg2.reshape(2, N, F), h, cf_w2[l],
                        cf_b2[l].reshape(1, H), lin_w[l], lin_b[l].reshape(1, H),
                        cf_w1[(l + 1) % L])
    out = _readout(h, batch.reshape(GRID_N, 1, TILE_N), out1_w,
                   out1_b.reshape(1, H), out2_w, out2_b.reshape(1, H))
    return out[:B]


# speedup vs baseline: 1.6720x; 1.0743x over previous
import math

import numpy as np
import jax
import jax.numpy as jnp
from jax import lax
from jax.experimental import pallas as pl
from jax.experimental.pallas import tpu as pltpu
from jax.experimental.pallas import tpu_sc as plsc

N = 10000
E = 640000
H = 128
F = 64
G = 50
L = 3
B = 100
CUTOFF = 10.0
LOG2 = math.log(2.0)

NC, NS = 2, 16
NW = NC * NS
EW = E // NW
K = 80
NCH = EW // K
RPS = N // NS

TILE_N = 1000
GRID_N = N // TILE_N
GP = 64
SPACING = CUTOFF / (G - 1)
COEFF = -0.5 / SPACING ** 2

TBL = 32768
TILE_T = 2048
GRID_T = TBL // TILE_T
WMAX = float(np.sqrt(27.0) * (1.0 + 1e-6))
DW = WMAX / (TBL - 1)
TILE_Q = 8000
GRID_Q = E // TILE_Q

_HIGH = jax.lax.Precision.HIGHEST

_mesh = plsc.VectorSubcoreMesh(core_axis_name="c", subcore_axis_name="s",
                               num_cores=NC, num_subcores=NS)

_PI = np.concatenate([np.concatenate([np.arange(g * 32, g * 32 + 32, 2),
                                      np.arange(g * 32 + 1, g * 32 + 32, 2)])
                      for g in range(2)]).astype(np.int32)


def _ssp(x):
    return jax.nn.softplus(x) - LOG2


def _geom_body(px_hbm, py_hbm, pz_hbm, src_hbm, dst_hbm, out_hbm,
               px_v, py_v, pz_v, src_v, dst_v, o_v):
    cid = lax.axis_index("c")
    sid = lax.axis_index("s")
    wid = sid * NC + cid
    base = wid * EW
    pltpu.sync_copy(px_hbm, px_v)
    pltpu.sync_copy(py_hbm, py_v)
    pltpu.sync_copy(pz_hbm, pz_v)
    pltpu.sync_copy(src_hbm.at[pl.ds(base, EW)], src_v)
    pltpu.sync_copy(dst_hbm.at[pl.ds(base, EW)], dst_v)

    @pl.loop(0, EW // 16)
    def _(i):
        s = src_v[pl.ds(i * 16, 16)]
        d = dst_v[pl.ds(i * 16, 16)]
        dx = plsc.load_gather(px_v, [s]) - plsc.load_gather(px_v, [d])
        dy = plsc.load_gather(py_v, [s]) - plsc.load_gather(py_v, [d])
        dz = plsc.load_gather(pz_v, [s]) - plsc.load_gather(pz_v, [d])
        o_v[pl.ds(i * 16, 16)] = dx * dx + dy * dy + dz * dz

    pltpu.sync_copy(o_v, out_hbm.at[pl.ds(base, EW)])


_geom = pl.kernel(
    _geom_body,
    out_type=jax.ShapeDtypeStruct((E,), jnp.float32),
    mesh=_mesh,
    scratch_types=[
        pltpu.VMEM((N,), jnp.float32),
        pltpu.VMEM((N,), jnp.float32),
        pltpu.VMEM((N,), jnp.float32),
        pltpu.VMEM((EW,), jnp.int32),
        pltpu.VMEM((EW,), jnp.int32),
        pltpu.VMEM((EW,), jnp.float32),
    ],
    compiler_params=pltpu.CompilerParams(needs_layout_passes=False),
)


def _gms_body(xf_hbm, tbl_hbm, src_hbm, dst_hbm, qidx_hbm, zero_hbm, out_hbm,
              src2_v, dst2_v, q2_v, gath0, gath1, wf0, wf1, msg0, msg1,
              agg_sh, gsem0, gsem1, wsem0, wsem1):
    cid = lax.axis_index("c")
    sid = lax.axis_index("s")
    wid = sid * NC + cid
    pltpu.sync_copy(zero_hbm.at[pl.ds(sid * RPS, RPS)],
                    agg_sh.at[pl.ds(sid * RPS, RPS)])
    pltpu.sync_copy(src_hbm.at[wid], src2_v)
    pltpu.sync_copy(dst_hbm.at[wid], dst2_v)
    pltpu.sync_copy(qidx_hbm.at[wid], q2_v)
    plsc.subcore_barrier()

    def start(cc, gath, wfb, gs, ws):
        pltpu.async_copy(xf_hbm.at[src2_v.at[cc]], gath, gs)
        pltpu.async_copy(tbl_hbm.at[q2_v.at[cc]], wfb, ws)

    def finish(cc, gath, wfb, msg, gs, ws):
        pltpu.make_async_copy(xf_hbm.at[src2_v.at[cc]], gath, gs).wait()
        pltpu.make_async_copy(tbl_hbm.at[q2_v.at[cc]], wfb, ws).wait()

        @plsc.parallel_loop(0, K, 1)
        def _(r):
            for g in range(2):
                ag, bg = plsc.unpack(gath[r, pl.ds(g * 32, 32)],
                                     format=plsc.PackFormat.INTERLEAVED,
                                     preferred_element_type=jnp.float32)
                aw, bw = plsc.unpack(wfb[r, pl.ds(g * 32, 32)],
                                     format=plsc.PackFormat.INTERLEAVED,
                                     preferred_element_type=jnp.float32)
                msg[r, pl.ds(g * 32, 16)] = ag * aw
                msg[r, pl.ds(g * 32 + 16, 16)] = bg * bw

        pltpu.sync_copy(msg, agg_sh.at[dst2_v.at[cc]], add=True)

    start(0, gath0, wf0, gsem0, wsem0)

    @pl.loop(0, NCH // 2)
    def _(j):
        cc0 = 2 * j
        start(cc0 + 1, gath1, wf1, gsem1, wsem1)
        finish(cc0, gath0, wf0, msg0, gsem0, wsem0)

        @pl.when(j < NCH // 2 - 1)
        def _():
            start(cc0 + 2, gath0, wf0, gsem0, wsem0)

        finish(cc0 + 1, gath1, wf1, msg1, gsem1, wsem1)

    plsc.subcore_barrier()
    pltpu.sync_copy(agg_sh.at[pl.ds(sid * RPS, RPS)],
                    out_hbm.at[pl.ds(cid * N + sid * RPS, RPS)])


_gms = pl.kernel(
    _gms_body,
    out_type=jax.ShapeDtypeStruct((2 * N, F), jnp.float32),
    mesh=_mesh,
    scratch_types=[
        pltpu.VMEM((NCH, K), jnp.int32),
        pltpu.VMEM((NCH, K), jnp.int32),
        pltpu.VMEM((NCH, K), jnp.int32),
        pltpu.VMEM((K, F), jnp.bfloat16),
        pltpu.VMEM((K, F), jnp.bfloat16),
        pltpu.VMEM((K, F), jnp.bfloat16),
        pltpu.VMEM((K, F), jnp.bfloat16),
        pltpu.VMEM((K, F), jnp.float32),
        pltpu.VMEM((K, F), jnp.float32),
        pltpu.VMEM_SHARED((N, F), jnp.float32),
        pltpu.SemaphoreType.DMA,
        pltpu.SemaphoreType.DMA,
        pltpu.SemaphoreType.DMA,
        pltpu.SemaphoreType.DMA,
    ],
    compiler_params=pltpu.CompilerParams(needs_layout_passes=False,
                                         use_tc_tiling_on_sc=False),
)


def _embed_body(z_ref, emb_ref, w1_ref, h_ref, xf_ref):
    zb = z_ref[0, 0, :]
    ids = lax.broadcasted_iota(jnp.int32, (TILE_N, 128), 1)
    oh = (zb[:, None] == ids).astype(jnp.float32)
    h = jnp.dot(oh, emb_ref[...], precision=_HIGH,
                preferred_element_type=jnp.float32)
    h_ref[...] = h
    xf_ref[...] = jnp.dot(h, w1_ref[...], precision=_HIGH,
                          preferred_element_type=jnp.float32
                          ).astype(jnp.bfloat16)


_embed = pl.pallas_call(
    _embed_body,
    grid=(GRID_N,),
    in_specs=[
        pl.BlockSpec((1, 1, TILE_N), lambda i: (i, 0, 0)),
        pl.BlockSpec((128, H), lambda i: (0, 0)),
        pl.BlockSpec((H, F), lambda i: (0, 0)),
    ],
    out_specs=[
        pl.BlockSpec((TILE_N, H), lambda i: (i, 0)),
        pl.BlockSpec((TILE_N, F), lambda i: (i, 0)),
    ],
    out_shape=[
        jax.ShapeDtypeStruct((N, H), jnp.float32),
        jax.ShapeDtypeStruct((N, F), jnp.bfloat16),
    ],
)


def _ftab_body(w1_ref, b1_ref, w2_ref, b2_ref, out_ref):
    t = pl.program_id(1)
    row = lax.broadcasted_iota(jnp.int32, (TILE_T, 1), 0) + t * TILE_T
    w = row.astype(jnp.float32) * DW
    offs = (lax.broadcasted_iota(jnp.int32, (1, GP), 1).astype(jnp.float32)
            * SPACING)
    attr = jnp.exp(COEFF * (w - offs) ** 2)
    tv = _ssp(jnp.dot(attr, w1_ref[0],
                      preferred_element_type=jnp.float32) + b1_ref[0])
    wf = jnp.dot(tv, w2_ref[0],
                 preferred_element_type=jnp.float32) + b2_ref[0]
    c = 0.5 * (jnp.cos(w * (math.pi / CUTOFF)) + 1.0)
    out_ref[0] = wf * c


_ftab = pl.pallas_call(
    _ftab_body,
    grid=(L, GRID_T),
    in_specs=[
        pl.BlockSpec((1, GP, F), lambda l, t: (l, 0, 0)),
        pl.BlockSpec((1, 1, F), lambda l, t: (l, 0, 0)),
        pl.BlockSpec((1, F, F), lambda l, t: (l, 0, 0)),
        pl.BlockSpec((1, 1, F), lambda l, t: (l, 0, 0)),
    ],
    out_specs=pl.BlockSpec((1, TILE_T, F), lambda l, t: (l, t, 0)),
    out_shape=jax.ShapeDtypeStruct((L, TBL, F), jnp.float32),
)


def _quant_body(wsq_ref, out_ref):
    w = jnp.sqrt(wsq_ref[0, 0, :] + 1e-12)
    q = jnp.round(w * (1.0 / DW)).astype(jnp.int32)
    out_ref[0, 0, :] = jnp.clip(q, 0, TBL - 1)


_quant = pl.pallas_call(
    _quant_body,
    grid=(GRID_Q,),
    in_specs=[pl.BlockSpec((1, 1, TILE_Q), lambda i: (i, 0, 0))],
    out_specs=pl.BlockSpec((1, 1, TILE_Q), lambda i: (i, 0, 0)),
    out_shape=jax.ShapeDtypeStruct((GRID_Q, 1, TILE_Q), jnp.int32),
)


def _update_body(agg_ref, h_ref, w2_ref, b2_ref, lw_ref, lb_ref, w1n_ref,
                 hout_ref, xfout_ref):
    agg = agg_ref[0] + agg_ref[1]
    x2 = jnp.dot(agg, w2_ref[...], precision=_HIGH,
                 preferred_element_type=jnp.float32) + b2_ref[...]
    hn = h_ref[...] + jnp.dot(_ssp(x2), lw_ref[...], precision=_HIGH,
                              preferred_element_type=jnp.float32) + lb_ref[...]
    hout_ref[...] = hn
    xfout_ref[...] = jnp.dot(hn, w1n_ref[...], precision=_HIGH,
                             preferred_element_type=jnp.float32
                             ).astype(jnp.bfloat16)


_update = pl.pallas_call(
    _update_body,
    grid=(GRID_N,),
    in_specs=[
        pl.BlockSpec((2, TILE_N, F), lambda i: (0, i, 0)),
        pl.BlockSpec((TILE_N, H), lambda i: (i, 0)),
        pl.BlockSpec((F, H), lambda i: (0, 0)),
        pl.BlockSpec((1, H), lambda i: (0, 0)),
        pl.BlockSpec((H, H), lambda i: (0, 0)),
        pl.BlockSpec((1, H), lambda i: (0, 0)),
        pl.BlockSpec((H, F), lambda i: (0, 0)),
    ],
    out_specs=[
        pl.BlockSpec((TILE_N, H), lambda i: (i, 0)),
        pl.BlockSpec((TILE_N, F), lambda i: (i, 0)),
    ],
    out_shape=[
        jax.ShapeDtypeStruct((N, H), jnp.float32),
        jax.ShapeDtypeStruct((N, F), jnp.bfloat16),
    ],
)


def _readout_body(h_ref, batch_ref, o1w_ref, o1b_ref, o2w_ref, o2b_ref,
                  out_ref, acc_s, cnt_s):
    i = pl.program_id(0)
    y = _ssp(jnp.dot(h_ref[...], o1w_ref[...], precision=_HIGH,
                     preferred_element_type=jnp.float32) + o1b_ref[...])
    y = jnp.dot(y, o2w_ref[...], precision=_HIGH,
                preferred_element_type=jnp.float32) + o2b_ref[...]
    bb = batch_ref[0, 0, :]
    rows = lax.broadcasted_iota(jnp.int32, (128, TILE_N), 0)
    mask = (rows == bb[None, :]).astype(jnp.float32)
    ms = jnp.dot(mask, y, precision=_HIGH, preferred_element_type=jnp.float32)
    mc = jnp.dot(mask, jnp.ones((TILE_N, 128), jnp.float32), precision=_HIGH,
                 preferred_element_type=jnp.float32)

    @pl.when(i == 0)
    def _():
        acc_s[...] = ms
        cnt_s[...] = mc

    @pl.when(i > 0)
    def _():
        acc_s[...] += ms
        cnt_s[...] += mc

    @pl.when(i == GRID_N - 1)
    def _():
        out_ref[...] = acc_s[...] / jnp.maximum(cnt_s[...], 1.0)


_readout = pl.pallas_call(
    _readout_body,
    grid=(GRID_N,),
    in_specs=[
        pl.BlockSpec((TILE_N, H), lambda i: (i, 0)),
        pl.BlockSpec((1, 1, TILE_N), lambda i: (i, 0, 0)),
        pl.BlockSpec((H, H), lambda i: (0, 0)),
        pl.BlockSpec((1, H), lambda i: (0, 0)),
        pl.BlockSpec((H, H), lambda i: (0, 0)),
        pl.BlockSpec((1, H), lambda i: (0, 0)),
    ],
    out_specs=pl.BlockSpec((128, H), lambda i: (0, 0)),
    out_shape=jax.ShapeDtypeStruct((128, H), jnp.float32),
    scratch_shapes=[
        pltpu.VMEM((128, H), jnp.float32),
        pltpu.VMEM((128, H), jnp.float32),
    ],
)


def kernel(z, pos, edge_index, batch, emb, mlp_w1, mlp_b1, mlp_w2, mlp_b2,
           cf_w1, cf_w2, cf_b2, lin_w, lin_b, out1_w, out1_b, out2_w, out2_b):
    src = edge_index[0]
    dst = edge_index[1]
    wsq = _geom(pos[:, 0], pos[:, 1], pos[:, 2], src, dst)
    embp = jnp.zeros((128, H), jnp.float32).at[:120].set(emb)
    h, xf = _embed(z.reshape(GRID_N, 1, TILE_N), embp, cf_w1[0])
    src2 = src.reshape(NW, NCH, K)
    dst2 = dst.reshape(NW, NCH, K)
    zeros = jnp.zeros((N, F), jnp.float32)
    qidx = _quant(wsq.reshape(GRID_Q, 1, TILE_Q)).reshape(NW, NCH, K)
    w1p = jnp.zeros((L, GP, F), jnp.float32).at[:, :G].set(mlp_w1)
    tbl = _ftab(w1p, mlp_b1.reshape(L, 1, F), mlp_w2,
                mlp_b2.reshape(L, 1, F)).astype(jnp.bfloat16)
    cf_w2p = cf_w2[:, _PI, :]
    for l in range(L):
        agg2 = _gms(xf, tbl[l], src2, dst2, qidx, zeros)
        h, xf = _update(agg2.reshape(2, N, F), h, cf_w2p[l],
                        cf_b2[l].reshape(1, H), lin_w[l], lin_b[l].reshape(1, H),
                        cf_w1[(l + 1) % L])
    out = _readout(h, batch.reshape(GRID_N, 1, TILE_N), out1_w,
                   out1_b.reshape(1, H), out2_w, out2_b.reshape(1, H))
    return out[:B]
